# baseline (speedup 1.0000x reference)
import functools

import jax
import jax.numpy as jnp
from jax import lax
from jax.experimental import pallas as pl
from jax.experimental.pallas import tpu as pltpu
from jax.experimental.pallas import tpu_sc as plsc


def _make_user_dot(V, D, B):
    info = plsc.get_sparse_core_info()
    NC, NS = info.num_cores, info.num_subcores
    NW = NC * NS
    assert B % (16 * NW) == 0
    b_per_w = B // NW
    mesh = plsc.VectorSubcoreMesh(core_axis_name="c", subcore_axis_name="s")

    @functools.partial(
        pl.kernel,
        mesh=mesh,
        compiler_params=pltpu.CompilerParams(use_tc_tiling_on_sc=True,
                                             needs_layout_passes=False),
        out_type=jax.ShapeDtypeStruct((NW, b_per_w), jnp.float32),
        scratch_types=[
            pltpu.VMEM((B,), jnp.int32),
            pltpu.VMEM((16,), jnp.float32),
            pltpu.VMEM((16, D, 128), jnp.float32),
            pltpu.VMEM((16, 16), jnp.float32),
            pltpu.VMEM((b_per_w,), jnp.float32),
            pltpu.SemaphoreType.DMA,
        ],
    )
    def dot_k(tableT_hbm, idx_hbm, wu_hbm, out_hbm, ids_v, wu_v, tiles_v,
              mat_v, res_v, sem):
        wid = lax.axis_index("s") * NC + lax.axis_index("c")
        base = wid * b_per_w
        pltpu.sync_copy(idx_hbm, ids_v)
        pltpu.sync_copy(wu_hbm, wu_v)
        wu_vec = wu_v[...]
        iota16 = lax.iota(jnp.int32, 16)
        zeros16 = jnp.zeros((16,), jnp.int32)

        for chunk in range(b_per_w // 16):
            ids16 = ids_v[pl.ds(base + chunk * 16, 16)]
            copies = []
            for j in range(16):
                tb = pl.multiple_of((ids16[j] // 128) * 128, 128)
                copies.append(pltpu.make_async_copy(
                    tableT_hbm.at[:, pl.ds(tb, 128)], tiles_v.at[j], sem))
            for c in copies:
                c.start()
            for c in copies:
                c.wait()
            for j in range(16):
                lane = ids16[j] - (ids16[j] // 128) * 128
                col = plsc.load_gather(tiles_v.at[j],
                                       [iota16, zeros16 + lane])
                plsc.store_scatter(mat_v, [iota16, zeros16 + j],
                                   col * wu_vec)
            acc = mat_v[0]
            for j in range(1, 16):
                acc = acc + mat_v[j]
            res_v[pl.ds(chunk * 16, 16)] = acc
        pltpu.sync_copy(res_v, out_hbm.at[wid])

    return dot_k


def _gru_body(ls_ref, ltT_ref, wih_ref, whh_ref, bih_ref, bhh_ref,
              fcw_ref, fcb_ref, out_ref, s_ref, acc_ref):
    L, B = ls_ref.shape
    H = acc_ref.shape[0]
    Gt = jnp.dot(wih_ref[...], ltT_ref[...],
                 preferred_element_type=jnp.float32) + bih_ref[...]
    whh = whh_ref[...]
    A = jnp.concatenate([
        jnp.concatenate([0.5 * whh[0:2 * H],
                         0.5 * (Gt[0:2 * H] + bhh_ref[0:2 * H])], axis=1),
        jnp.concatenate([0.5 * whh[2 * H:3 * H],
                         jnp.broadcast_to(0.5 * bhh_ref[2 * H:3 * H],
                                          (H, 3))], axis=1),
        jnp.concatenate([0.5 * whh[2 * H:3 * H],
                         Gt[2 * H:3 * H] + jnp.broadcast_to(
                             0.5 * bhh_ref[2 * H:3 * H], (H, 3))], axis=1),
    ], axis=0)
    iota3 = lax.broadcasted_iota(jnp.int32, (3, 1), 0)

    s_ref[...] = jnp.zeros_like(s_ref)
    acc_ref[...] = jnp.zeros_like(acc_ref)

    def substep(t, h):
        ls_row = ls_ref[pl.ds(t, 1), :]
        s_ref[H:H + 3, :] = jnp.equal(ls_row, iota3).astype(jnp.float32)
        gates = jnp.dot(A, s_ref[...],
                        preferred_element_type=jnp.float32)
        rz = jnp.tanh(gates[0:2 * H])
        rp = rz[0:H]
        zp = rz[H:2 * H]
        n = jnp.tanh(gates[3 * H:4 * H] + rp * gates[2 * H:3 * H])
        h_new = 0.5 * ((h + n) + zp * (h - n))
        s_ref[0:H, :] = h_new
        return h_new

    def step(i, h):
        h1 = substep(2 * i, h)
        h2 = substep(2 * i + 1, h1)
        acc_ref[...] += h1 + h2
        return h2

    lax.fori_loop(0, L // 2, step, jnp.zeros((H, B), jnp.float32),
                  unroll=4)

    seq_feat = acc_ref[...] * (1.0 / L)
    ws = fcw_ref[H:2 * H, :]
    contrib = ws * seq_feat
    out_ref[...] = jnp.sum(contrib, axis=0, keepdims=True) + fcb_ref[...]


def _gru_call(ls_t, ltT, W_ih, W_hh, b_ih2, b_hh2, fcwT, fcb2):
    L, B = ls_t.shape
    H = W_hh.shape[1]
    return pl.pallas_call(
        _gru_body,
        out_shape=jax.ShapeDtypeStruct((1, B), jnp.float32),
        scratch_shapes=[
            pltpu.VMEM((H + 3, B), jnp.float32),
            pltpu.VMEM((H, B), jnp.float32),
        ],
    )(ls_t, ltT, W_ih, W_hh, b_ih2, b_hh2, fcwT, fcb2)


def kernel(user_id, launch_seq, user_table, launch_table, W_ih, W_hh, b_ih,
           b_hh, fc_W, fc_b):
    B, L = launch_seq.shape
    V, E = user_table.shape
    H = W_hh.shape[1]

    user_dot = _make_user_dot(V, E, B)
    u_row = user_dot(user_table.T, user_id.astype(jnp.int32),
                     fc_W[0, 0:H])

    ls_t = launch_seq.T
    ltT = launch_table.T
    b_ih2 = b_ih.reshape(3 * H, 1)
    b_hh2 = b_hh.reshape(3 * H, 1)
    fcwT = fc_W.reshape(2 * H, 1)
    fcb2 = fc_b.reshape(1, 1)

    seq_row = _gru_call(ls_t, ltT, W_ih, W_hh, b_ih2, b_hh2, fcwT, fcb2)
    return (seq_row.reshape(B, 1) + u_row.reshape(B, 1)).astype(jnp.float32)


# speedup vs baseline: 1.0068x; 1.0068x over previous
import functools

import jax
import jax.numpy as jnp
from jax import lax
from jax.experimental import pallas as pl
from jax.experimental.pallas import tpu as pltpu
from jax.experimental.pallas import tpu_sc as plsc


def _make_user_dot(V, D, B):
    info = plsc.get_sparse_core_info()
    NC, NS = info.num_cores, info.num_subcores
    NW = NC * NS
    assert B % (16 * NW) == 0
    b_per_w = B // NW
    mesh = plsc.VectorSubcoreMesh(core_axis_name="c", subcore_axis_name="s")

    @functools.partial(
        pl.kernel,
        mesh=mesh,
        compiler_params=pltpu.CompilerParams(use_tc_tiling_on_sc=True,
                                             needs_layout_passes=False),
        out_type=jax.ShapeDtypeStruct((NW, b_per_w), jnp.float32),
        scratch_types=[
            pltpu.VMEM((B,), jnp.int32),
            pltpu.VMEM((32,), jnp.float32),
            pltpu.VMEM((16, D, 128), jnp.float32),
            pltpu.VMEM((16, 16), jnp.float32),
            pltpu.VMEM((b_per_w,), jnp.float32),
            pltpu.SemaphoreType.DMA,
        ],
    )
    def dot_k(tableT_hbm, idx_hbm, wu_hbm, out_hbm, ids_v, wu_v, tiles_v,
              mat_v, res_v, sem):
        wid = lax.axis_index("s") * NC + lax.axis_index("c")
        base = wid * b_per_w
        pltpu.sync_copy(idx_hbm, ids_v)
        pltpu.sync_copy(wu_hbm, wu_v)
        wu_vec = wu_v[pl.ds(0, 16)]
        iota16 = lax.iota(jnp.int32, 16)
        zeros16 = jnp.zeros((16,), jnp.int32)

        for chunk in range(b_per_w // 16):
            ids16 = ids_v[pl.ds(base + chunk * 16, 16)]
            copies = []
            for j in range(16):
                tb = pl.multiple_of((ids16[j] // 128) * 128, 128)
                copies.append(pltpu.make_async_copy(
                    tableT_hbm.at[:, pl.ds(tb, 128)], tiles_v.at[j], sem))
            for c in copies:
                c.start()
            for c in copies:
                c.wait()
            for j in range(16):
                lane = ids16[j] - (ids16[j] // 128) * 128
                col = plsc.load_gather(tiles_v.at[j],
                                       [iota16, zeros16 + lane])
                plsc.store_scatter(mat_v, [iota16, zeros16 + j],
                                   col * wu_vec)
            acc = mat_v[0]
            for j in range(1, 16):
                acc = acc + mat_v[j]
            res_v[pl.ds(chunk * 16, 16)] = acc
        pltpu.sync_copy(res_v, out_hbm.at[wid])

    return dot_k


def _gru_body(ls_ref, ltT_ref, wih_ref, whh_ref, bih_ref, bhh_ref,
              fcw_ref, fcb_ref, out_ref, s_ref, acc_ref):
    L, B = ls_ref.shape
    H = acc_ref.shape[0]
    Gt = jnp.dot(wih_ref[...], ltT_ref[...],
                 preferred_element_type=jnp.float32) + bih_ref[...]
    whh = whh_ref[...]
    A = jnp.concatenate([
        jnp.concatenate([0.5 * whh[0:2 * H],
                         0.5 * (Gt[0:2 * H] + bhh_ref[0:2 * H])], axis=1),
        jnp.concatenate([0.5 * whh[2 * H:3 * H],
                         jnp.broadcast_to(0.5 * bhh_ref[2 * H:3 * H],
                                          (H, 3))], axis=1),
        jnp.concatenate([0.5 * whh[2 * H:3 * H],
                         Gt[2 * H:3 * H] + jnp.broadcast_to(
                             0.5 * bhh_ref[2 * H:3 * H], (H, 3))], axis=1),
    ], axis=0)
    iota3 = lax.broadcasted_iota(jnp.int32, (3, 1), 0)

    s_ref[...] = jnp.zeros_like(s_ref)
    acc_ref[...] = jnp.zeros_like(acc_ref)

    def substep(t, h):
        ls_row = ls_ref[pl.ds(t, 1), :]
        s_ref[H:H + 3, :] = jnp.equal(ls_row, iota3).astype(jnp.float32)
        gates = jnp.dot(A, s_ref[...],
                        preferred_element_type=jnp.float32)
        rz = jnp.tanh(gates[0:2 * H])
        rp = rz[0:H]
        zp = rz[H:2 * H]
        n = jnp.tanh(gates[3 * H:4 * H] + rp * gates[2 * H:3 * H])
        h_new = 0.5 * ((h + n) + zp * (h - n))
        s_ref[0:H, :] = h_new
        return h_new

    def step(i, h):
        h1 = substep(2 * i, h)
        h2 = substep(2 * i + 1, h1)
        acc_ref[...] += h1 + h2
        return h2

    lax.fori_loop(0, L // 2, step, jnp.zeros((H, B), jnp.float32),
                  unroll=4)

    seq_feat = acc_ref[...] * (1.0 / L)
    ws = fcw_ref[H:2 * H, :]
    contrib = ws * seq_feat
    out_ref[...] = jnp.sum(contrib, axis=0, keepdims=True) + fcb_ref[...]


def _gru_call(ls_t, ltT, W_ih, W_hh, b_ih2, b_hh2, fcwT, fcb2):
    L, B = ls_t.shape
    H = W_hh.shape[1]
    return pl.pallas_call(
        _gru_body,
        out_shape=jax.ShapeDtypeStruct((1, B), jnp.float32),
        scratch_shapes=[
            pltpu.VMEM((H + 3, B), jnp.float32),
            pltpu.VMEM((H, B), jnp.float32),
        ],
    )(ls_t, ltT, W_ih, W_hh, b_ih2, b_hh2, fcwT, fcb2)


def kernel(user_id, launch_seq, user_table, launch_table, W_ih, W_hh, b_ih,
           b_hh, fc_W, fc_b):
    B, L = launch_seq.shape
    V, E = user_table.shape
    H = W_hh.shape[1]

    user_dot = _make_user_dot(V, E, B)
    u_row = user_dot(user_table.T, user_id.astype(jnp.int32),
                     fc_W.reshape(2 * H))

    ls_t = launch_seq.T
    ltT = launch_table.T
    b_ih2 = b_ih.reshape(3 * H, 1)
    b_hh2 = b_hh.reshape(3 * H, 1)
    fcwT = fc_W.reshape(2 * H, 1)
    fcb2 = fc_b.reshape(1, 1)

    seq_row = _gru_call(ls_t, ltT, W_ih, W_hh, b_ih2, b_hh2, fcwT, fcb2)
    return (seq_row.reshape(B, 1) + u_row.reshape(B, 1)).astype(jnp.float32)


# speedup vs baseline: 1.0482x; 1.0412x over previous
import functools

import jax
import jax.numpy as jnp
from jax import lax
from jax.experimental import pallas as pl
from jax.experimental.pallas import tpu as pltpu
from jax.experimental.pallas import tpu_sc as plsc


def _make_user_dot(V, D, B):
    info = plsc.get_sparse_core_info()
    NC, NS = info.num_cores, info.num_subcores
    NW = NC * NS
    assert B % (16 * NW) == 0
    b_per_w = B // NW
    mesh = plsc.VectorSubcoreMesh(core_axis_name="c", subcore_axis_name="s")

    @functools.partial(
        pl.kernel,
        mesh=mesh,
        compiler_params=pltpu.CompilerParams(use_tc_tiling_on_sc=True,
                                             needs_layout_passes=False),
        out_type=jax.ShapeDtypeStruct((NW, b_per_w), jnp.float32),
        scratch_types=[
            pltpu.VMEM((B,), jnp.int32),
            pltpu.VMEM((32,), jnp.float32),
            pltpu.VMEM((16, D, 128), jnp.float32),
            pltpu.VMEM((16, 16), jnp.float32),
            pltpu.VMEM((b_per_w,), jnp.float32),
            pltpu.SemaphoreType.DMA,
        ],
    )
    def dot_k(tableT_hbm, idx_hbm, wu_hbm, out_hbm, ids_v, wu_v, tiles_v,
              mat_v, res_v, sem):
        wid = lax.axis_index("s") * NC + lax.axis_index("c")
        base = wid * b_per_w
        pltpu.sync_copy(idx_hbm, ids_v)
        pltpu.sync_copy(wu_hbm, wu_v)
        wu_vec = wu_v[pl.ds(0, 16)]
        iota16 = lax.iota(jnp.int32, 16)
        zeros16 = jnp.zeros((16,), jnp.int32)

        for chunk in range(b_per_w // 16):
            ids16 = ids_v[pl.ds(base + chunk * 16, 16)]
            copies = []
            for j in range(16):
                tb = pl.multiple_of((ids16[j] // 128) * 128, 128)
                copies.append(pltpu.make_async_copy(
                    tableT_hbm.at[:, pl.ds(tb, 128)], tiles_v.at[j], sem))
            for c in copies:
                c.start()
            for c in copies:
                c.wait()
            for j in range(16):
                lane = ids16[j] - (ids16[j] // 128) * 128
                col = plsc.load_gather(tiles_v.at[j],
                                       [iota16, zeros16 + lane])
                plsc.store_scatter(mat_v, [iota16, zeros16 + j],
                                   col * wu_vec)
            acc = mat_v[0]
            for j in range(1, 16):
                acc = acc + mat_v[j]
            res_v[pl.ds(chunk * 16, 16)] = acc
        pltpu.sync_copy(res_v, out_hbm.at[wid])

    return dot_k


def _gru_body(ls_ref, ltT_ref, wih_ref, whh_ref, bih_ref, bhh_ref,
              fcw_ref, fcb_ref, out_ref, s_ref, acc_ref):
    L, B = ls_ref.shape
    H = acc_ref.shape[0]
    Gt = jnp.dot(wih_ref[...], ltT_ref[...],
                 preferred_element_type=jnp.float32) + bih_ref[...]
    whh = whh_ref[...]
    A = jnp.concatenate([
        jnp.concatenate([0.5 * whh[0:2 * H],
                         0.5 * (Gt[0:2 * H] + bhh_ref[0:2 * H])], axis=1),
        jnp.concatenate([0.5 * whh[2 * H:3 * H],
                         jnp.broadcast_to(0.5 * bhh_ref[2 * H:3 * H],
                                          (H, 3))], axis=1),
        jnp.concatenate([0.5 * whh[2 * H:3 * H],
                         Gt[2 * H:3 * H] + jnp.broadcast_to(
                             0.5 * bhh_ref[2 * H:3 * H], (H, 3))], axis=1),
    ], axis=0)
    iota3 = lax.broadcasted_iota(jnp.int32, (3, 1), 0)

    s_ref[...] = jnp.zeros_like(s_ref)
    acc_ref[...] = jnp.zeros_like(acc_ref)

    def substep(t, h):
        ls_row = ls_ref[pl.ds(t, 1), :]
        s_ref[H:H + 3, :] = jnp.equal(ls_row, iota3).astype(jnp.float32)
        gates = jnp.dot(A, s_ref[...],
                        preferred_element_type=jnp.float32)
        rz = jnp.tanh(gates[0:2 * H])
        rp = rz[0:H]
        zp = rz[H:2 * H]
        n = jnp.tanh(gates[3 * H:4 * H] + rp * gates[2 * H:3 * H])
        h_new = 0.5 * ((h + n) + zp * (h - n))
        s_ref[0:H, :] = h_new
        return h_new

    def step(i, h):
        h1 = substep(2 * i, h)
        h2 = substep(2 * i + 1, h1)
        acc_ref[...] += h1 + h2
        return h2

    lax.fori_loop(0, L // 2, step, jnp.zeros((H, B), jnp.float32),
                  unroll=8)

    seq_feat = acc_ref[...] * (1.0 / L)
    ws = fcw_ref[H:2 * H, :]
    contrib = ws * seq_feat
    out_ref[...] = jnp.sum(contrib, axis=0, keepdims=True) + fcb_ref[...]


def _gru_call(ls_t, ltT, W_ih, W_hh, b_ih2, b_hh2, fcwT, fcb2):
    L, B = ls_t.shape
    H = W_hh.shape[1]
    return pl.pallas_call(
        _gru_body,
        out_shape=jax.ShapeDtypeStruct((1, B), jnp.float32),
        scratch_shapes=[
            pltpu.VMEM((H + 3, B), jnp.float32),
            pltpu.VMEM((H, B), jnp.float32),
        ],
    )(ls_t, ltT, W_ih, W_hh, b_ih2, b_hh2, fcwT, fcb2)


def kernel(user_id, launch_seq, user_table, launch_table, W_ih, W_hh, b_ih,
           b_hh, fc_W, fc_b):
    B, L = launch_seq.shape
    V, E = user_table.shape
    H = W_hh.shape[1]

    user_dot = _make_user_dot(V, E, B)
    u_row = user_dot(user_table.T, user_id.astype(jnp.int32),
                     fc_W.reshape(2 * H))

    ls_t = launch_seq.T
    ltT = launch_table.T
    b_ih2 = b_ih.reshape(3 * H, 1)
    b_hh2 = b_hh.reshape(3 * H, 1)
    fcwT = fc_W.reshape(2 * H, 1)
    fcb2 = fc_b.reshape(1, 1)

    seq_row = _gru_call(ls_t, ltT, W_ih, W_hh, b_ih2, b_hh2, fcwT, fcb2)
    return (seq_row.reshape(B, 1) + u_row.reshape(B, 1)).astype(jnp.float32)


# speedup vs baseline: 1.0707x; 1.0215x over previous
import functools

import jax
import jax.numpy as jnp
from jax import lax
from jax.experimental import pallas as pl
from jax.experimental.pallas import tpu as pltpu
from jax.experimental.pallas import tpu_sc as plsc


def _make_user_dot(V, D, B):
    info = plsc.get_sparse_core_info()
    NC, NS = info.num_cores, info.num_subcores
    NW = NC * NS
    assert B % (16 * NW) == 0
    b_per_w = B // NW
    mesh = plsc.VectorSubcoreMesh(core_axis_name="c", subcore_axis_name="s")

    @functools.partial(
        pl.kernel,
        mesh=mesh,
        compiler_params=pltpu.CompilerParams(use_tc_tiling_on_sc=True,
                                             needs_layout_passes=False),
        out_type=jax.ShapeDtypeStruct((NW, b_per_w), jnp.float32),
        scratch_types=[
            pltpu.VMEM((B,), jnp.int32),
            pltpu.VMEM((32,), jnp.float32),
            pltpu.VMEM((16, D, 128), jnp.float32),
            pltpu.VMEM((16, 16), jnp.float32),
            pltpu.VMEM((b_per_w,), jnp.float32),
            pltpu.SemaphoreType.DMA,
        ],
    )
    def dot_k(tableT_hbm, idx_hbm, wu_hbm, out_hbm, ids_v, wu_v, tiles_v,
              mat_v, res_v, sem):
        wid = lax.axis_index("s") * NC + lax.axis_index("c")
        base = wid * b_per_w
        pltpu.sync_copy(idx_hbm, ids_v)
        pltpu.sync_copy(wu_hbm, wu_v)
        wu_vec = wu_v[pl.ds(0, 16)]
        iota16 = lax.iota(jnp.int32, 16)
        zeros16 = jnp.zeros((16,), jnp.int32)

        for chunk in range(b_per_w // 16):
            ids16 = ids_v[pl.ds(base + chunk * 16, 16)]
            copies = []
            for j in range(16):
                tb = pl.multiple_of((ids16[j] // 128) * 128, 128)
                copies.append(pltpu.make_async_copy(
                    tableT_hbm.at[:, pl.ds(tb, 128)], tiles_v.at[j], sem))
            for c in copies:
                c.start()
            for c in copies:
                c.wait()
            for j in range(16):
                lane = ids16[j] - (ids16[j] // 128) * 128
                col = plsc.load_gather(tiles_v.at[j],
                                       [iota16, zeros16 + lane])
                plsc.store_scatter(mat_v, [iota16, zeros16 + j],
                                   col * wu_vec)
            acc = mat_v[0]
            for j in range(1, 16):
                acc = acc + mat_v[j]
            res_v[pl.ds(chunk * 16, 16)] = acc
        pltpu.sync_copy(res_v, out_hbm.at[wid])

    return dot_k


def _gru_body(ls_ref, ltT_ref, wih_ref, whh_ref, bih_ref, bhh_ref,
              fcw_ref, fcb_ref, out_ref, s_ref, acc_ref):
    L, B = ls_ref.shape
    H = acc_ref.shape[0]
    Gt = jnp.dot(wih_ref[...], ltT_ref[...],
                 preferred_element_type=jnp.float32) + bih_ref[...]
    whh = whh_ref[...]
    A = jnp.concatenate([
        jnp.concatenate([0.5 * whh[0:2 * H],
                         0.5 * (Gt[0:2 * H] + bhh_ref[0:2 * H])], axis=1),
        jnp.concatenate([0.5 * whh[2 * H:3 * H],
                         jnp.broadcast_to(0.5 * bhh_ref[2 * H:3 * H],
                                          (H, 3))], axis=1),
        jnp.concatenate([0.5 * whh[2 * H:3 * H],
                         Gt[2 * H:3 * H] + jnp.broadcast_to(
                             0.5 * bhh_ref[2 * H:3 * H], (H, 3))], axis=1),
    ], axis=0)
    iota3 = lax.broadcasted_iota(jnp.int32, (3, 1), 0)

    s_ref[...] = jnp.zeros_like(s_ref)
    acc_ref[...] = jnp.zeros_like(acc_ref)

    def substep(t, h):
        ls_row = ls_ref[pl.ds(t, 1), :]
        s_ref[H:H + 3, :] = jnp.equal(ls_row, iota3).astype(jnp.float32)
        gates = jnp.dot(A, s_ref[...],
                        preferred_element_type=jnp.float32)
        rz = jnp.tanh(gates[0:2 * H])
        rp = rz[0:H]
        zp = rz[H:2 * H]
        n = jnp.tanh(gates[3 * H:4 * H] + rp * gates[2 * H:3 * H])
        h_new = 0.5 * ((h + n) + zp * (h - n))
        s_ref[0:H, :] = h_new
        return h_new

    def step(i, h):
        h1 = substep(2 * i, h)
        h2 = substep(2 * i + 1, h1)
        acc_ref[...] += h1 + h2
        return h2

    lax.fori_loop(0, L // 2, step, jnp.zeros((H, B), jnp.float32),
                  unroll=16)

    seq_feat = acc_ref[...] * (1.0 / L)
    ws = fcw_ref[H:2 * H, :]
    contrib = ws * seq_feat
    out_ref[...] = jnp.sum(contrib, axis=0, keepdims=True) + fcb_ref[...]


def _gru_call(ls_t, ltT, W_ih, W_hh, b_ih2, b_hh2, fcwT, fcb2):
    L, B = ls_t.shape
    H = W_hh.shape[1]
    return pl.pallas_call(
        _gru_body,
        out_shape=jax.ShapeDtypeStruct((1, B), jnp.float32),
        scratch_shapes=[
            pltpu.VMEM((H + 3, B), jnp.float32),
            pltpu.VMEM((H, B), jnp.float32),
        ],
    )(ls_t, ltT, W_ih, W_hh, b_ih2, b_hh2, fcwT, fcb2)


def kernel(user_id, launch_seq, user_table, launch_table, W_ih, W_hh, b_ih,
           b_hh, fc_W, fc_b):
    B, L = launch_seq.shape
    V, E = user_table.shape
    H = W_hh.shape[1]

    user_dot = _make_user_dot(V, E, B)
    u_row = user_dot(user_table.T, user_id.astype(jnp.int32),
                     fc_W.reshape(2 * H))

    ls_t = launch_seq.T
    ltT = launch_table.T
    b_ih2 = b_ih.reshape(3 * H, 1)
    b_hh2 = b_hh.reshape(3 * H, 1)
    fcwT = fc_W.reshape(2 * H, 1)
    fcb2 = fc_b.reshape(1, 1)

    seq_row = _gru_call(ls_t, ltT, W_ih, W_hh, b_ih2, b_hh2, fcwT, fcb2)
    return (seq_row.reshape(B, 1) + u_row.reshape(B, 1)).astype(jnp.float32)


# speedup vs baseline: 1.0839x; 1.0123x over previous
import functools

import jax
import jax.numpy as jnp
from jax import lax
from jax.experimental import pallas as pl
from jax.experimental.pallas import tpu as pltpu
from jax.experimental.pallas import tpu_sc as plsc


def _make_user_dot(V, D, B):
    info = plsc.get_sparse_core_info()
    NC, NS = info.num_cores, info.num_subcores
    NW = NC * NS
    assert B % (16 * NW) == 0
    b_per_w = B // NW
    mesh = plsc.VectorSubcoreMesh(core_axis_name="c", subcore_axis_name="s")

    @functools.partial(
        pl.kernel,
        mesh=mesh,
        compiler_params=pltpu.CompilerParams(use_tc_tiling_on_sc=True,
                                             needs_layout_passes=False),
        out_type=jax.ShapeDtypeStruct((NW, b_per_w), jnp.float32),
        scratch_types=[
            pltpu.VMEM((B,), jnp.int32),
            pltpu.VMEM((32,), jnp.float32),
            pltpu.VMEM((16, D, 128), jnp.float32),
            pltpu.VMEM((16, 16), jnp.float32),
            pltpu.VMEM((b_per_w,), jnp.float32),
            pltpu.SemaphoreType.DMA,
        ],
    )
    def dot_k(tableT_hbm, idx_hbm, wu_hbm, out_hbm, ids_v, wu_v, tiles_v,
              mat_v, res_v, sem):
        wid = lax.axis_index("s") * NC + lax.axis_index("c")
        base = wid * b_per_w
        pltpu.sync_copy(idx_hbm, ids_v)
        pltpu.sync_copy(wu_hbm, wu_v)
        wu_vec = wu_v[pl.ds(0, 16)]
        iota16 = lax.iota(jnp.int32, 16)
        zeros16 = jnp.zeros((16,), jnp.int32)

        for chunk in range(b_per_w // 16):
            ids16 = ids_v[pl.ds(base + chunk * 16, 16)]
            copies = []
            for j in range(16):
                tb = pl.multiple_of((ids16[j] // 128) * 128, 128)
                copies.append(pltpu.make_async_copy(
                    tableT_hbm.at[:, pl.ds(tb, 128)], tiles_v.at[j], sem))
            for c in copies:
                c.start()
            for c in copies:
                c.wait()
            for j in range(16):
                lane = ids16[j] - (ids16[j] // 128) * 128
                col = plsc.load_gather(tiles_v.at[j],
                                       [iota16, zeros16 + lane])
                plsc.store_scatter(mat_v, [iota16, zeros16 + j],
                                   col * wu_vec)
            acc = mat_v[0]
            for j in range(1, 16):
                acc = acc + mat_v[j]
            res_v[pl.ds(chunk * 16, 16)] = acc
        pltpu.sync_copy(res_v, out_hbm.at[wid])

    return dot_k


def _gru_body(ls_ref, ltT_ref, wih_ref, whh_ref, bih_ref, bhh_ref,
              fcw_ref, fcb_ref, out_ref, s_ref, acc_ref):
    L, B = ls_ref.shape
    H = acc_ref.shape[0]
    Gt = jnp.dot(wih_ref[...], ltT_ref[...],
                 preferred_element_type=jnp.float32) + bih_ref[...]
    whh = whh_ref[...]
    A = jnp.concatenate([
        jnp.concatenate([0.5 * whh[0:2 * H],
                         0.5 * (Gt[0:2 * H] + bhh_ref[0:2 * H])], axis=1),
        jnp.concatenate([0.5 * whh[2 * H:3 * H],
                         jnp.broadcast_to(0.5 * bhh_ref[2 * H:3 * H],
                                          (H, 3))], axis=1),
        jnp.concatenate([0.5 * whh[2 * H:3 * H],
                         Gt[2 * H:3 * H] + jnp.broadcast_to(
                             0.5 * bhh_ref[2 * H:3 * H], (H, 3))], axis=1),
    ], axis=0)
    iota3 = lax.broadcasted_iota(jnp.int32, (3, 1), 0)

    s_ref[...] = jnp.zeros_like(s_ref)
    acc_ref[...] = jnp.zeros_like(acc_ref)

    def substep(t, h):
        ls_row = ls_ref[pl.ds(t, 1), :]
        s_ref[H:H + 3, :] = jnp.equal(ls_row, iota3).astype(jnp.float32)
        gates = jnp.dot(A, s_ref[...],
                        preferred_element_type=jnp.float32)
        rz = jnp.tanh(gates[0:2 * H])
        rp = rz[0:H]
        zp = rz[H:2 * H]
        n = jnp.tanh(gates[3 * H:4 * H] + rp * gates[2 * H:3 * H])
        h_new = 0.5 * ((h + n) + zp * (h - n))
        s_ref[0:H, :] = h_new
        return h_new

    def step(i, h):
        h1 = substep(2 * i, h)
        h2 = substep(2 * i + 1, h1)
        acc_ref[...] += h1 + h2
        return h2

    lax.fori_loop(0, L // 2, step, jnp.zeros((H, B), jnp.float32),
                  unroll=25)

    seq_feat = acc_ref[...] * (1.0 / L)
    ws = fcw_ref[H:2 * H, :]
    contrib = ws * seq_feat
    out_ref[...] = jnp.sum(contrib, axis=0, keepdims=True) + fcb_ref[...]


def _gru_call(ls_t, ltT, W_ih, W_hh, b_ih2, b_hh2, fcwT, fcb2):
    L, B = ls_t.shape
    H = W_hh.shape[1]
    return pl.pallas_call(
        _gru_body,
        out_shape=jax.ShapeDtypeStruct((1, B), jnp.float32),
        scratch_shapes=[
            pltpu.VMEM((H + 3, B), jnp.float32),
            pltpu.VMEM((H, B), jnp.float32),
        ],
    )(ls_t, ltT, W_ih, W_hh, b_ih2, b_hh2, fcwT, fcb2)


def kernel(user_id, launch_seq, user_table, launch_table, W_ih, W_hh, b_ih,
           b_hh, fc_W, fc_b):
    B, L = launch_seq.shape
    V, E = user_table.shape
    H = W_hh.shape[1]

    user_dot = _make_user_dot(V, E, B)
    u_row = user_dot(user_table.T, user_id.astype(jnp.int32),
                     fc_W.reshape(2 * H))

    ls_t = launch_seq.T
    ltT = launch_table.T
    b_ih2 = b_ih.reshape(3 * H, 1)
    b_hh2 = b_hh.reshape(3 * H, 1)
    fcwT = fc_W.reshape(2 * H, 1)
    fcb2 = fc_b.reshape(1, 1)

    seq_row = _gru_call(ls_t, ltT, W_ih, W_hh, b_ih2, b_hh2, fcwT, fcb2)
    return (seq_row.reshape(B, 1) + u_row.reshape(B, 1)).astype(jnp.float32)


# speedup vs baseline: 1.0904x; 1.0059x over previous
import functools

import jax
import jax.numpy as jnp
from jax import lax
from jax.experimental import pallas as pl
from jax.experimental.pallas import tpu as pltpu
from jax.experimental.pallas import tpu_sc as plsc


def _make_user_dot(V, D, B):
    info = plsc.get_sparse_core_info()
    NC, NS = info.num_cores, info.num_subcores
    NW = NC * NS
    assert B % (16 * NW) == 0
    b_per_w = B // NW
    mesh = plsc.VectorSubcoreMesh(core_axis_name="c", subcore_axis_name="s")

    @functools.partial(
        pl.kernel,
        mesh=mesh,
        compiler_params=pltpu.CompilerParams(use_tc_tiling_on_sc=True,
                                             needs_layout_passes=False),
        out_type=jax.ShapeDtypeStruct((NW, b_per_w), jnp.float32),
        scratch_types=[
            pltpu.VMEM((B,), jnp.int32),
            pltpu.VMEM((32,), jnp.float32),
            pltpu.VMEM((16, D, 128), jnp.float32),
            pltpu.VMEM((16, 16), jnp.float32),
            pltpu.VMEM((b_per_w,), jnp.float32),
            pltpu.SemaphoreType.DMA,
        ],
    )
    def dot_k(tableT_hbm, idx_hbm, wu_hbm, out_hbm, ids_v, wu_v, tiles_v,
              mat_v, res_v, sem):
        wid = lax.axis_index("s") * NC + lax.axis_index("c")
        base = wid * b_per_w
        pltpu.sync_copy(idx_hbm, ids_v)
        pltpu.sync_copy(wu_hbm, wu_v)
        wu_vec = wu_v[pl.ds(0, 16)]
        iota16 = lax.iota(jnp.int32, 16)
        zeros16 = jnp.zeros((16,), jnp.int32)

        for chunk in range(b_per_w // 16):
            ids16 = ids_v[pl.ds(base + chunk * 16, 16)]
            copies = []
            for j in range(16):
                tb = pl.multiple_of((ids16[j] // 128) * 128, 128)
                copies.append(pltpu.make_async_copy(
                    tableT_hbm.at[:, pl.ds(tb, 128)], tiles_v.at[j], sem))
            for c in copies:
                c.start()
            for c in copies:
                c.wait()
            for j in range(16):
                lane = ids16[j] - (ids16[j] // 128) * 128
                col = plsc.load_gather(tiles_v.at[j],
                                       [iota16, zeros16 + lane])
                plsc.store_scatter(mat_v, [iota16, zeros16 + j],
                                   col * wu_vec)
            acc = mat_v[0]
            for j in range(1, 16):
                acc = acc + mat_v[j]
            res_v[pl.ds(chunk * 16, 16)] = acc
        pltpu.sync_copy(res_v, out_hbm.at[wid])

    return dot_k


def _gru_body(ls_ref, ltT_ref, wih_ref, whh_ref, bih_ref, bhh_ref,
              fcw_ref, fcb_ref, out_ref, s_ref, acc_ref):
    L, B = ls_ref.shape
    H = acc_ref.shape[0]
    Gt = jnp.dot(wih_ref[...], ltT_ref[...],
                 preferred_element_type=jnp.float32) + bih_ref[...]
    whh = whh_ref[...]
    A = jnp.concatenate([
        jnp.concatenate([0.5 * whh[0:2 * H],
                         0.5 * (Gt[0:2 * H] + bhh_ref[0:2 * H])], axis=1),
        jnp.concatenate([0.5 * whh[2 * H:3 * H],
                         jnp.broadcast_to(0.5 * bhh_ref[2 * H:3 * H],
                                          (H, 3))], axis=1),
        jnp.concatenate([0.5 * whh[2 * H:3 * H],
                         Gt[2 * H:3 * H] + jnp.broadcast_to(
                             0.5 * bhh_ref[2 * H:3 * H], (H, 3))], axis=1),
    ], axis=0)
    iota3 = lax.broadcasted_iota(jnp.int32, (3, 1), 0)

    s_ref[...] = jnp.zeros_like(s_ref)
    acc_ref[...] = jnp.zeros_like(acc_ref)

    def substep(t, h):
        ls_row = ls_ref[pl.ds(t, 1), :]
        s_ref[H:H + 3, :] = jnp.equal(ls_row, iota3).astype(jnp.float32)
        gates = jnp.dot(A, s_ref[...],
                        preferred_element_type=jnp.float32)
        rz = jnp.tanh(gates[0:2 * H])
        rp = rz[0:H]
        zp = rz[H:2 * H]
        n = jnp.tanh(gates[3 * H:4 * H] + rp * gates[2 * H:3 * H])
        h_new = 0.5 * ((h + n) + zp * (h - n))
        s_ref[0:H, :] = h_new
        return h_new

    def step(i, h):
        h1 = substep(2 * i, h)
        h2 = substep(2 * i + 1, h1)
        acc_ref[...] += h1 + h2
        return h2

    lax.fori_loop(0, L // 2, step, jnp.zeros((H, B), jnp.float32),
                  unroll=50)

    seq_feat = acc_ref[...] * (1.0 / L)
    ws = fcw_ref[H:2 * H, :]
    contrib = ws * seq_feat
    out_ref[...] = jnp.sum(contrib, axis=0, keepdims=True) + fcb_ref[...]


def _gru_call(ls_t, ltT, W_ih, W_hh, b_ih2, b_hh2, fcwT, fcb2):
    L, B = ls_t.shape
    H = W_hh.shape[1]
    return pl.pallas_call(
        _gru_body,
        out_shape=jax.ShapeDtypeStruct((1, B), jnp.float32),
        scratch_shapes=[
            pltpu.VMEM((H + 3, B), jnp.float32),
            pltpu.VMEM((H, B), jnp.float32),
        ],
    )(ls_t, ltT, W_ih, W_hh, b_ih2, b_hh2, fcwT, fcb2)


def kernel(user_id, launch_seq, user_table, launch_table, W_ih, W_hh, b_ih,
           b_hh, fc_W, fc_b):
    B, L = launch_seq.shape
    V, E = user_table.shape
    H = W_hh.shape[1]

    user_dot = _make_user_dot(V, E, B)
    u_row = user_dot(user_table.T, user_id.astype(jnp.int32),
                     fc_W.reshape(2 * H))

    ls_t = launch_seq.T
    ltT = launch_table.T
    b_ih2 = b_ih.reshape(3 * H, 1)
    b_hh2 = b_hh.reshape(3 * H, 1)
    fcwT = fc_W.reshape(2 * H, 1)
    fcb2 = fc_b.reshape(1, 1)

    seq_row = _gru_call(ls_t, ltT, W_ih, W_hh, b_ih2, b_hh2, fcwT, fcb2)
    return (seq_row.reshape(B, 1) + u_row.reshape(B, 1)).astype(jnp.float32)
